# Initial kernel scaffold; baseline (speedup 1.0000x reference)
import jax
import jax.numpy as jnp
from jax.experimental import pallas as pl


def kernel(x, edge_index, emb0, W0, b0, att0, bias0, W1, b1, att1, bias1):
    raise NotImplementedError("write your pallas kernel here")


# speedup vs baseline: 15.7478x; 15.7478x over previous
import functools
import jax
import jax.numpy as jnp
from jax.experimental import pallas as pl
from jax.experimental.pallas import tpu as pltpu

N = 10000
E = 320000
EMB = 128
BLK = 400


def _combine_matmul_kernel(hbar_ref, ideg_ref, od_ref, p0_ref, p1_ref, bias_ref,
                           wt_ref, bm_ref, g_ref, h1_ref, v1_ref):
    hbar = hbar_ref[...]
    ideg = ideg_ref[...]
    od = od_ref[...]
    comb = ideg * hbar + hbar / od + p0_ref[...] + p1_ref[...] + bias_ref[...]
    g = jnp.maximum(comb, 0.0)
    g_ref[...] = g
    h1 = jnp.dot(g, wt_ref[...], preferred_element_type=jnp.float32) + bm_ref[...]
    h1_ref[...] = h1
    v1_ref[...] = h1 / od


def _combine_kernel(hbar_ref, ideg_ref, od_ref, p0_ref, p1_ref, bias_ref, out_ref):
    hbar = hbar_ref[...]
    out_ref[...] = (ideg_ref[...] * hbar + hbar / od_ref[...]
                    + p0_ref[...] + p1_ref[...] + bias_ref[...])


def _row_spec():
    return pl.BlockSpec((BLK, EMB), lambda i: (i, 0))


def _col_spec():
    return pl.BlockSpec((BLK, 1), lambda i: (i, 0))


def _full_spec(shape):
    return pl.BlockSpec(shape, lambda i: tuple(0 for _ in shape))


def _combine_matmul(hbar, ideg, od, p0, p1, bias, wt, bm):
    grid = N // BLK
    return pl.pallas_call(
        _combine_matmul_kernel,
        grid=(grid,),
        in_specs=[_row_spec(), _col_spec(), _col_spec(), _row_spec(), _row_spec(),
                  _full_spec((1, EMB)), _full_spec((EMB, EMB)), _full_spec((1, EMB))],
        out_specs=[_row_spec(), _row_spec(), _row_spec()],
        out_shape=[jax.ShapeDtypeStruct((N, EMB), jnp.float32)] * 3,
    )(hbar, ideg, od, p0, p1, bias, wt, bm)


def _combine(hbar, ideg, od, p0, p1, bias):
    grid = N // BLK
    return pl.pallas_call(
        _combine_kernel,
        grid=(grid,),
        in_specs=[_row_spec(), _col_spec(), _col_spec(), _row_spec(), _row_spec(),
                  _full_spec((1, EMB))],
        out_specs=_row_spec(),
        out_shape=jax.ShapeDtypeStruct((N, EMB), jnp.float32),
    )(hbar, ideg, od, p0, p1, bias)


def kernel(x, edge_index, emb0, W0, b0, att0, bias0, W1, b1, att1, bias1):
    src = edge_index[0]
    dst = edge_index[1]
    ones = jnp.ones((E,), jnp.float32)
    od = (jax.ops.segment_sum(ones, src, num_segments=N) + 1.0)[:, None]
    ideg = (jax.ops.segment_sum(ones, dst, num_segments=N) + 1.0)[:, None]

    Wm0t = (0.5 * (W0[:EMB] + W0[EMB:])).T
    bm0 = (0.5 * (b0[:EMB] + b0[EMB:]))[None, :]
    Wm1t = (0.5 * (W1[:EMB] + W1[EMB:])).T
    bm1 = (0.5 * (b1[:EMB] + b1[EMB:]))[None, :]

    rows2 = emb0 @ Wm0t + bm0
    hbar0 = rows2[x]
    v0 = hbar0 / od
    s0 = jax.ops.segment_sum(v0[src], dst, num_segments=N)
    zero = jnp.zeros((N, EMB), jnp.float32)

    _, h1, v1 = _combine_matmul(hbar0, ideg, od, s0, zero, bias0[None, :], Wm1t, bm1)
    s1 = jax.ops.segment_sum(v1[src], dst, num_segments=N)
    return _combine(h1, ideg, od, s1, zero, bias1[None, :])


# speedup vs baseline: 69.5842x; 4.4187x over previous
import functools
import jax
import jax.numpy as jnp
from jax import lax
from jax.experimental import pallas as pl
from jax.experimental.pallas import tpu as pltpu
from jax.experimental.pallas import tpu_sc as plsc

N = 10000
NPAD = 10240
E = 320000
EMB = 128
BLK = 400
NC = 2
NS = 16
EPT = E // (NC * NS)
CH = 128
NFULL = EPT // CH
REM = EPT - NFULL * CH
SLAB = NPAD // NS


def _sc_mesh():
    return plsc.VectorSubcoreMesh(core_axis_name="c", subcore_axis_name="s")


def _deg_body(src_hbm, dst_hbm, z128_hbm, pay_hbm, out_hbm,
              deg_s, idx_v, idx16_v, pay_v):
    c = lax.axis_index("c")
    s = lax.axis_index("s")
    base = (c * NS + s) * EPT
    slab = s * SLAB
    pltpu.sync_copy(z128_hbm.at[pl.ds(slab, SLAB)], deg_s.at[pl.ds(slab, SLAB)])
    pltpu.sync_copy(pay_hbm, pay_v)
    plsc.subcore_barrier()

    def body(i, carry):
        off = base + i * CH
        pltpu.sync_copy(src_hbm.at[pl.ds(off, CH)], idx_v)
        pltpu.sync_copy(pay_v.at[0], deg_s.at[idx_v], add=True)
        pltpu.sync_copy(dst_hbm.at[pl.ds(off, CH)], idx_v)
        pltpu.sync_copy(pay_v.at[1], deg_s.at[idx_v], add=True)
        return carry

    lax.fori_loop(0, NFULL, body, 0)
    off = base + NFULL * CH
    pltpu.sync_copy(src_hbm.at[pl.ds(off, REM)], idx16_v)
    pltpu.sync_copy(pay_v.at[0, pl.ds(0, REM)], deg_s.at[idx16_v], add=True)
    pltpu.sync_copy(dst_hbm.at[pl.ds(off, REM)], idx16_v)
    pltpu.sync_copy(pay_v.at[1, pl.ds(0, REM)], deg_s.at[idx16_v], add=True)
    plsc.subcore_barrier()
    pltpu.sync_copy(deg_s.at[pl.ds(slab, SLAB)], out_hbm.at[c, pl.ds(slab, SLAB)])


def _degrees(src, dst, z128):
    col = jnp.arange(EMB)[None, :]
    pay = jnp.stack([
        jnp.where(col == 0, 1.0, 0.0) * jnp.ones((CH, 1), jnp.float32),
        jnp.where(col == 64, 1.0, 0.0) * jnp.ones((CH, 1), jnp.float32),
    ]).astype(jnp.float32)
    kfn = pl.kernel(
        _deg_body,
        out_type=jax.ShapeDtypeStruct((NC, NPAD, EMB), jnp.float32),
        mesh=_sc_mesh(),
        scratch_types=[
            pltpu.VMEM_SHARED((NPAD, EMB), jnp.float32),
            pltpu.VMEM((CH,), jnp.int32),
            pltpu.VMEM((REM,), jnp.int32),
            pltpu.VMEM((2, CH, EMB), jnp.float32),
        ],
    )
    return kfn(src, dst, z128, pay)


def _pass_body(src_hbm, dst_hbm, v_hbm, z128_hbm, out_hbm,
               out_s, sidx_v, didx_v, idx16a_v, idx16b_v, rows_v, rows16_v, sem):
    c = lax.axis_index("c")
    s = lax.axis_index("s")
    base = (c * NS + s) * EPT
    slab = s * SLAB
    pltpu.sync_copy(z128_hbm.at[pl.ds(slab, SLAB)], out_s.at[pl.ds(slab, SLAB)])
    plsc.subcore_barrier()

    def body(i, carry):
        off = base + i * CH
        pltpu.sync_copy(src_hbm.at[pl.ds(off, CH)], sidx_v)
        pltpu.async_copy(v_hbm.at[sidx_v], rows_v, sem).wait()
        pltpu.sync_copy(dst_hbm.at[pl.ds(off, CH)], didx_v)
        pltpu.sync_copy(rows_v, out_s.at[didx_v], add=True)
        return carry

    lax.fori_loop(0, NFULL, body, 0)
    off = base + NFULL * CH
    pltpu.sync_copy(src_hbm.at[pl.ds(off, REM)], idx16a_v)
    pltpu.async_copy(v_hbm.at[idx16a_v], rows16_v, sem).wait()
    pltpu.sync_copy(dst_hbm.at[pl.ds(off, REM)], idx16b_v)
    pltpu.sync_copy(rows16_v, out_s.at[idx16b_v], add=True)
    plsc.subcore_barrier()
    pltpu.sync_copy(out_s.at[pl.ds(slab, SLAB)], out_hbm.at[c, pl.ds(slab, SLAB)])


def _edge_pass(src, dst, v, z128):
    kfn = pl.kernel(
        _pass_body,
        out_type=jax.ShapeDtypeStruct((NC, NPAD, EMB), jnp.float32),
        mesh=_sc_mesh(),
        scratch_types=[
            pltpu.VMEM_SHARED((NPAD, EMB), jnp.float32),
            pltpu.VMEM((CH,), jnp.int32),
            pltpu.VMEM((CH,), jnp.int32),
            pltpu.VMEM((REM,), jnp.int32),
            pltpu.VMEM((REM,), jnp.int32),
            pltpu.VMEM((CH, EMB), jnp.float32),
            pltpu.VMEM((REM, EMB), jnp.float32),
            pltpu.SemaphoreType.DMA,
        ],
    )
    return kfn(src, dst, v, z128)


def _prep0_kernel(x_ref, emb0_ref, w0t_ref, bm0_ref, ip0_ref, ip1_ref,
                  op0_ref, op1_ref, hbar_ref, v0_ref, ideg_ref, od_ref):
    rows2 = jnp.dot(emb0_ref[...], w0t_ref[...],
                    preferred_element_type=jnp.float32) + bm0_ref[...]
    hbar = jnp.where(x_ref[...] == 0, rows2[0:1, :], rows2[1:2, :])
    od = op0_ref[...] + op1_ref[...] + 1.0
    hbar_ref[...] = hbar
    v0_ref[...] = hbar / od
    ideg_ref[...] = ip0_ref[...] + ip1_ref[...] + 1.0
    od_ref[...] = od


def _mid_kernel(hbar_ref, v0_ref, ideg_ref, od_ref, p0_ref, p1_ref, bias_ref,
                w1t_ref, bm1_ref, h1_ref, v1_ref):
    comb = (ideg_ref[...] * hbar_ref[...] + v0_ref[...]
            + p0_ref[...] + p1_ref[...] + bias_ref[...])
    g = jnp.maximum(comb, 0.0)
    h1 = jnp.dot(g, w1t_ref[...], preferred_element_type=jnp.float32) + bm1_ref[...]
    h1_ref[...] = h1
    v1_ref[...] = h1 / od_ref[...]


def _final_kernel(h1_ref, v1_ref, ideg_ref, p0_ref, p1_ref, bias_ref, out_ref):
    out_ref[...] = (ideg_ref[...] * h1_ref[...] + v1_ref[...]
                    + p0_ref[...] + p1_ref[...] + bias_ref[...])


def _row_spec():
    return pl.BlockSpec((BLK, EMB), lambda i: (i, 0))


def _col_spec():
    return pl.BlockSpec((BLK, 1), lambda i: (i, 0))


def _full_spec(shape):
    return pl.BlockSpec(shape, lambda i: tuple(0 for _ in shape))


def _prep0(x2, emb0, w0t, bm0, ip0, ip1, op0, op1):
    return pl.pallas_call(
        _prep0_kernel,
        grid=(N // BLK,),
        in_specs=[_col_spec(), _full_spec((2, EMB)), _full_spec((EMB, EMB)),
                  _full_spec((1, EMB)), _col_spec(), _col_spec(),
                  _col_spec(), _col_spec()],
        out_specs=[_row_spec(), _row_spec(), _col_spec(), _col_spec()],
        out_shape=[jax.ShapeDtypeStruct((N, EMB), jnp.float32),
                   jax.ShapeDtypeStruct((N, EMB), jnp.float32),
                   jax.ShapeDtypeStruct((N, 1), jnp.float32),
                   jax.ShapeDtypeStruct((N, 1), jnp.float32)],
    )(x2, emb0, w0t, bm0, ip0, ip1, op0, op1)


def _mid(hbar, v0, ideg, od, p0, p1, bias, w1t, bm1):
    return pl.pallas_call(
        _mid_kernel,
        grid=(N // BLK,),
        in_specs=[_row_spec(), _row_spec(), _col_spec(), _col_spec(),
                  _row_spec(), _row_spec(), _full_spec((1, EMB)),
                  _full_spec((EMB, EMB)), _full_spec((1, EMB))],
        out_specs=[_row_spec(), _row_spec()],
        out_shape=[jax.ShapeDtypeStruct((N, EMB), jnp.float32),
                   jax.ShapeDtypeStruct((N, EMB), jnp.float32)],
    )(hbar, v0, ideg, od, p0, p1, bias, w1t, bm1)


def _final(h1, v1, ideg, p0, p1, bias):
    return pl.pallas_call(
        _final_kernel,
        grid=(N // BLK,),
        in_specs=[_row_spec(), _row_spec(), _col_spec(), _row_spec(),
                  _row_spec(), _full_spec((1, EMB))],
        out_specs=_row_spec(),
        out_shape=jax.ShapeDtypeStruct((N, EMB), jnp.float32),
    )(h1, v1, ideg, p0, p1, bias)


def kernel(x, edge_index, emb0, W0, b0, att0, bias0, W1, b1, att1, bias1):
    src = edge_index[0]
    dst = edge_index[1]
    x2 = x[:, None]

    w0t = (0.5 * (W0[:EMB] + W0[EMB:])).T
    bm0 = (0.5 * (b0[:EMB] + b0[EMB:]))[None, :]
    w1t = (0.5 * (W1[:EMB] + W1[EMB:])).T
    bm1 = (0.5 * (b1[:EMB] + b1[EMB:]))[None, :]

    z128 = jnp.zeros((NPAD, EMB), jnp.float32)
    degp = _degrees(src, dst, z128)
    op0 = degp[0, :N, 0:1]
    op1 = degp[1, :N, 0:1]
    ip0 = degp[0, :N, 64:65]
    ip1 = degp[1, :N, 64:65]

    hbar0, v0, ideg, od = _prep0(x2, emb0, w0t, bm0, ip0, ip1, op0, op1)
    part0 = _edge_pass(src, dst, v0, z128)
    h1, v1 = _mid(hbar0, v0, ideg, od, part0[0, :N], part0[1, :N],
                  bias0[None, :], w1t, bm1)
    part1 = _edge_pass(src, dst, v1, z128)
    return _final(h1, v1, ideg, part1[0, :N], part1[1, :N], bias1[None, :])


# speedup vs baseline: 91.7003x; 1.3178x over previous
import functools
import jax
import jax.numpy as jnp
from jax import lax
from jax.experimental import pallas as pl
from jax.experimental.pallas import tpu as pltpu
from jax.experimental.pallas import tpu_sc as plsc

N = 10000
NPAD = 10240
E = 320000
EMB = 128
BLK = 400
NC = 2
NS = 16
EPT = E // (NC * NS)
CH = 128
NFULL = EPT // CH
REM = EPT - NFULL * CH
SLAB = NPAD // NS


def _sc_mesh():
    return plsc.VectorSubcoreMesh(core_axis_name="c", subcore_axis_name="s")


def _deg_body(src_hbm, dst_hbm, z128_hbm, pay_hbm, out_hbm,
              deg_s, idx_v, idx16_v, pay_v):
    c = lax.axis_index("c")
    s = lax.axis_index("s")
    base = (c * NS + s) * EPT
    slab = s * SLAB
    pltpu.sync_copy(z128_hbm.at[pl.ds(slab, SLAB)], deg_s.at[pl.ds(slab, SLAB)])
    pltpu.sync_copy(pay_hbm, pay_v)
    plsc.subcore_barrier()

    def body(i, carry):
        off = base + i * CH
        pltpu.sync_copy(src_hbm.at[pl.ds(off, CH)], idx_v)
        pltpu.sync_copy(pay_v.at[0], deg_s.at[idx_v], add=True)
        pltpu.sync_copy(dst_hbm.at[pl.ds(off, CH)], idx_v)
        pltpu.sync_copy(pay_v.at[1], deg_s.at[idx_v], add=True)
        return carry

    lax.fori_loop(0, NFULL, body, 0)
    off = base + NFULL * CH
    pltpu.sync_copy(src_hbm.at[pl.ds(off, REM)], idx16_v)
    pltpu.sync_copy(pay_v.at[0, pl.ds(0, REM)], deg_s.at[idx16_v], add=True)
    pltpu.sync_copy(dst_hbm.at[pl.ds(off, REM)], idx16_v)
    pltpu.sync_copy(pay_v.at[1, pl.ds(0, REM)], deg_s.at[idx16_v], add=True)
    plsc.subcore_barrier()
    pltpu.sync_copy(deg_s.at[pl.ds(slab, SLAB)], out_hbm.at[c, pl.ds(slab, SLAB)])


def _degrees(src, dst, z128):
    col = jnp.arange(EMB)[None, :]
    pay = jnp.stack([
        jnp.where(col == 0, 1.0, 0.0) * jnp.ones((CH, 1), jnp.float32),
        jnp.where(col == 64, 1.0, 0.0) * jnp.ones((CH, 1), jnp.float32),
    ]).astype(jnp.float32)
    kfn = pl.kernel(
        _deg_body,
        out_type=jax.ShapeDtypeStruct((NC, NPAD, EMB), jnp.float32),
        mesh=_sc_mesh(),
        scratch_types=[
            pltpu.VMEM_SHARED((NPAD, EMB), jnp.float32),
            pltpu.VMEM((CH,), jnp.int32),
            pltpu.VMEM((REM,), jnp.int32),
            pltpu.VMEM((2, CH, EMB), jnp.float32),
        ],
    )
    return kfn(src, dst, z128, pay)


def _pass_body(src_hbm, dst_hbm, v_hbm, z128_hbm, out_hbm,
               out_s, sidx0_v, sidx1_v, didx0_v, didx1_v,
               idx16a_v, idx16b_v, rows0_v, rows1_v, rows16_v,
               sem0, sem1, sem16):
    c = lax.axis_index("c")
    s = lax.axis_index("s")
    base = (c * NS + s) * EPT
    slab = s * SLAB
    pltpu.sync_copy(z128_hbm.at[pl.ds(slab, SLAB)], out_s.at[pl.ds(slab, SLAB)])
    plsc.subcore_barrier()

    pltpu.sync_copy(src_hbm.at[pl.ds(base, CH)], sidx0_v)
    pltpu.async_copy(v_hbm.at[sidx0_v], rows0_v, sem0)

    def body(i, carry):
        off1 = base + (2 * i + 1) * CH
        pltpu.sync_copy(src_hbm.at[pl.ds(off1, CH)], sidx1_v)
        pltpu.async_copy(v_hbm.at[sidx1_v], rows1_v, sem1)
        pltpu.make_async_copy(v_hbm.at[sidx0_v], rows0_v, sem0).wait()
        pltpu.sync_copy(dst_hbm.at[pl.ds(off1 - CH, CH)], didx0_v)
        pltpu.sync_copy(rows0_v, out_s.at[didx0_v], add=True)

        @pl.when(i < NFULL // 2 - 1)
        def _():
            off2 = off1 + CH
            pltpu.sync_copy(src_hbm.at[pl.ds(off2, CH)], sidx0_v)
            pltpu.async_copy(v_hbm.at[sidx0_v], rows0_v, sem0)

        pltpu.make_async_copy(v_hbm.at[sidx1_v], rows1_v, sem1).wait()
        pltpu.sync_copy(dst_hbm.at[pl.ds(off1, CH)], didx1_v)
        pltpu.sync_copy(rows1_v, out_s.at[didx1_v], add=True)
        return carry

    lax.fori_loop(0, NFULL // 2, body, 0)
    off = base + NFULL * CH
    pltpu.sync_copy(src_hbm.at[pl.ds(off, REM)], idx16a_v)
    pltpu.async_copy(v_hbm.at[idx16a_v], rows16_v, sem16).wait()
    pltpu.sync_copy(dst_hbm.at[pl.ds(off, REM)], idx16b_v)
    pltpu.sync_copy(rows16_v, out_s.at[idx16b_v], add=True)
    plsc.subcore_barrier()
    pltpu.sync_copy(out_s.at[pl.ds(slab, SLAB)], out_hbm.at[c, pl.ds(slab, SLAB)])


def _edge_pass(src, dst, v, z128):
    kfn = pl.kernel(
        _pass_body,
        out_type=jax.ShapeDtypeStruct((NC, NPAD, EMB), jnp.float32),
        mesh=_sc_mesh(),
        scratch_types=[
            pltpu.VMEM_SHARED((NPAD, EMB), jnp.float32),
            pltpu.VMEM((CH,), jnp.int32),
            pltpu.VMEM((CH,), jnp.int32),
            pltpu.VMEM((CH,), jnp.int32),
            pltpu.VMEM((CH,), jnp.int32),
            pltpu.VMEM((REM,), jnp.int32),
            pltpu.VMEM((REM,), jnp.int32),
            pltpu.VMEM((CH, EMB), jnp.float32),
            pltpu.VMEM((CH, EMB), jnp.float32),
            pltpu.VMEM((REM, EMB), jnp.float32),
            pltpu.SemaphoreType.DMA,
            pltpu.SemaphoreType.DMA,
            pltpu.SemaphoreType.DMA,
        ],
    )
    return kfn(src, dst, v, z128)


def _prep0_kernel(x_ref, emb0_ref, w0t_ref, bm0_ref, ip0_ref, ip1_ref,
                  op0_ref, op1_ref, hbar_ref, v0_ref, ideg_ref, od_ref):
    rows2 = jnp.dot(emb0_ref[...], w0t_ref[...],
                    preferred_element_type=jnp.float32) + bm0_ref[...]
    hbar = jnp.where(x_ref[...] == 0, rows2[0:1, :], rows2[1:2, :])
    od = op0_ref[...] + op1_ref[...] + 1.0
    hbar_ref[...] = hbar
    v0_ref[...] = hbar / od
    ideg_ref[...] = ip0_ref[...] + ip1_ref[...] + 1.0
    od_ref[...] = od


def _mid_kernel(hbar_ref, v0_ref, ideg_ref, od_ref, p0_ref, p1_ref, bias_ref,
                w1t_ref, bm1_ref, h1_ref, v1_ref):
    comb = (ideg_ref[...] * hbar_ref[...] + v0_ref[...]
            + p0_ref[...] + p1_ref[...] + bias_ref[...])
    g = jnp.maximum(comb, 0.0)
    h1 = jnp.dot(g, w1t_ref[...], preferred_element_type=jnp.float32) + bm1_ref[...]
    h1_ref[...] = h1
    v1_ref[...] = h1 / od_ref[...]


def _final_kernel(h1_ref, v1_ref, ideg_ref, p0_ref, p1_ref, bias_ref, out_ref):
    out_ref[...] = (ideg_ref[...] * h1_ref[...] + v1_ref[...]
                    + p0_ref[...] + p1_ref[...] + bias_ref[...])


def _row_spec():
    return pl.BlockSpec((BLK, EMB), lambda i: (i, 0))


def _col_spec():
    return pl.BlockSpec((BLK, 1), lambda i: (i, 0))


def _full_spec(shape):
    return pl.BlockSpec(shape, lambda i: tuple(0 for _ in shape))


def _prep0(x2, emb0, w0t, bm0, ip0, ip1, op0, op1):
    return pl.pallas_call(
        _prep0_kernel,
        grid=(N // BLK,),
        in_specs=[_col_spec(), _full_spec((2, EMB)), _full_spec((EMB, EMB)),
                  _full_spec((1, EMB)), _col_spec(), _col_spec(),
                  _col_spec(), _col_spec()],
        out_specs=[_row_spec(), _row_spec(), _col_spec(), _col_spec()],
        out_shape=[jax.ShapeDtypeStruct((N, EMB), jnp.float32),
                   jax.ShapeDtypeStruct((N, EMB), jnp.float32),
                   jax.ShapeDtypeStruct((N, 1), jnp.float32),
                   jax.ShapeDtypeStruct((N, 1), jnp.float32)],
    )(x2, emb0, w0t, bm0, ip0, ip1, op0, op1)


def _mid(hbar, v0, ideg, od, p0, p1, bias, w1t, bm1):
    return pl.pallas_call(
        _mid_kernel,
        grid=(N // BLK,),
        in_specs=[_row_spec(), _row_spec(), _col_spec(), _col_spec(),
                  _row_spec(), _row_spec(), _full_spec((1, EMB)),
                  _full_spec((EMB, EMB)), _full_spec((1, EMB))],
        out_specs=[_row_spec(), _row_spec()],
        out_shape=[jax.ShapeDtypeStruct((N, EMB), jnp.float32),
                   jax.ShapeDtypeStruct((N, EMB), jnp.float32)],
    )(hbar, v0, ideg, od, p0, p1, bias, w1t, bm1)


def _final(h1, v1, ideg, p0, p1, bias):
    return pl.pallas_call(
        _final_kernel,
        grid=(N // BLK,),
        in_specs=[_row_spec(), _row_spec(), _col_spec(), _row_spec(),
                  _row_spec(), _full_spec((1, EMB))],
        out_specs=_row_spec(),
        out_shape=jax.ShapeDtypeStruct((N, EMB), jnp.float32),
    )(h1, v1, ideg, p0, p1, bias)


def kernel(x, edge_index, emb0, W0, b0, att0, bias0, W1, b1, att1, bias1):
    src = edge_index[0]
    dst = edge_index[1]
    x2 = x[:, None]

    w0t = (0.5 * (W0[:EMB] + W0[EMB:])).T
    bm0 = (0.5 * (b0[:EMB] + b0[EMB:]))[None, :]
    w1t = (0.5 * (W1[:EMB] + W1[EMB:])).T
    bm1 = (0.5 * (b1[:EMB] + b1[EMB:]))[None, :]

    z128 = jnp.zeros((NPAD, EMB), jnp.float32)
    degp = _degrees(src, dst, z128)
    op0 = degp[0, :N, 0:1]
    op1 = degp[1, :N, 0:1]
    ip0 = degp[0, :N, 64:65]
    ip1 = degp[1, :N, 64:65]

    hbar0, v0, ideg, od = _prep0(x2, emb0, w0t, bm0, ip0, ip1, op0, op1)
    part0 = _edge_pass(src, dst, v0, z128)
    h1, v1 = _mid(hbar0, v0, ideg, od, part0[0, :N], part0[1, :N],
                  bias0[None, :], w1t, bm1)
    part1 = _edge_pass(src, dst, v1, z128)
    return _final(h1, v1, ideg, part1[0, :N], part1[1, :N], bias1[None, :])


# speedup vs baseline: 96.1044x; 1.0480x over previous
import functools
import jax
import jax.numpy as jnp
from jax import lax
from jax.experimental import pallas as pl
from jax.experimental.pallas import tpu as pltpu
from jax.experimental.pallas import tpu_sc as plsc

N = 10000
NPAD = 10240
E = 320000
EMB = 128
BLK = 400
NC = 2
NS = 16
EPT = E // (NC * NS)
CH = 128
NFULL = EPT // CH
REM = EPT - NFULL * CH
SLAB = NPAD // NS


def _sc_mesh():
    return plsc.VectorSubcoreMesh(core_axis_name="c", subcore_axis_name="s")


def _deg_body(src_hbm, dst_hbm, z128_hbm, pay_hbm, out_hbm,
              deg_s, idx_v, idx16_v, pay_v):
    c = lax.axis_index("c")
    s = lax.axis_index("s")
    base = (c * NS + s) * EPT
    slab = s * SLAB
    pltpu.sync_copy(z128_hbm.at[pl.ds(slab, SLAB)], deg_s.at[pl.ds(slab, SLAB)])
    pltpu.sync_copy(pay_hbm, pay_v)
    plsc.subcore_barrier()

    def body(i, carry):
        off = base + i * CH
        pltpu.sync_copy(src_hbm.at[pl.ds(off, CH)], idx_v)
        pltpu.sync_copy(pay_v.at[0], deg_s.at[idx_v], add=True)
        pltpu.sync_copy(dst_hbm.at[pl.ds(off, CH)], idx_v)
        pltpu.sync_copy(pay_v.at[1], deg_s.at[idx_v], add=True)
        return carry

    lax.fori_loop(0, NFULL, body, 0)
    off = base + NFULL * CH
    pltpu.sync_copy(src_hbm.at[pl.ds(off, REM)], idx16_v)
    pltpu.sync_copy(pay_v.at[0, pl.ds(0, REM)], deg_s.at[idx16_v], add=True)
    pltpu.sync_copy(dst_hbm.at[pl.ds(off, REM)], idx16_v)
    pltpu.sync_copy(pay_v.at[1, pl.ds(0, REM)], deg_s.at[idx16_v], add=True)
    plsc.subcore_barrier()
    pltpu.sync_copy(deg_s.at[pl.ds(slab, SLAB)], out_hbm.at[c, pl.ds(slab, SLAB)])


def _degrees(src, dst, z128):
    col = jnp.arange(EMB)[None, :]
    pay = jnp.stack([
        jnp.where(col == 0, 1.0, 0.0) * jnp.ones((CH, 1), jnp.float32),
        jnp.where(col == 64, 1.0, 0.0) * jnp.ones((CH, 1), jnp.float32),
    ]).astype(jnp.float32)
    kfn = pl.kernel(
        _deg_body,
        out_type=jax.ShapeDtypeStruct((NC, NPAD, EMB), jnp.float32),
        mesh=_sc_mesh(),
        scratch_types=[
            pltpu.VMEM_SHARED((NPAD, EMB), jnp.float32),
            pltpu.VMEM((CH,), jnp.int32),
            pltpu.VMEM((REM,), jnp.int32),
            pltpu.VMEM((2, CH, EMB), jnp.float32),
        ],
    )
    return kfn(src, dst, z128, pay)


def _pass_body(src_hbm, dst_hbm, v_hbm, z128_hbm, out_hbm,
               out_s, sidx0_v, sidx1_v, didx0_v, didx1_v,
               idx16a_v, idx16b_v, rows0_v, rows1_v, rows16_v,
               sem0, sem1, sem16):
    c = lax.axis_index("c")
    s = lax.axis_index("s")
    base = (c * NS + s) * EPT
    slab = s * SLAB
    pltpu.sync_copy(z128_hbm.at[pl.ds(slab, SLAB)], out_s.at[pl.ds(slab, SLAB)])
    plsc.subcore_barrier()

    pltpu.sync_copy(src_hbm.at[pl.ds(base, CH)], sidx0_v)
    pltpu.async_copy(v_hbm.at[sidx0_v], rows0_v, sem0)

    def body(i, carry):
        off1 = base + (2 * i + 1) * CH
        pltpu.sync_copy(src_hbm.at[pl.ds(off1, CH)], sidx1_v)
        pltpu.async_copy(v_hbm.at[sidx1_v], rows1_v, sem1)
        pltpu.make_async_copy(v_hbm.at[sidx0_v], rows0_v, sem0).wait()
        pltpu.sync_copy(dst_hbm.at[pl.ds(off1 - CH, CH)], didx0_v)
        pltpu.sync_copy(rows0_v, out_s.at[didx0_v], add=True)

        @pl.when(i < NFULL // 2 - 1)
        def _():
            off2 = off1 + CH
            pltpu.sync_copy(src_hbm.at[pl.ds(off2, CH)], sidx0_v)
            pltpu.async_copy(v_hbm.at[sidx0_v], rows0_v, sem0)

        pltpu.make_async_copy(v_hbm.at[sidx1_v], rows1_v, sem1).wait()
        pltpu.sync_copy(dst_hbm.at[pl.ds(off1, CH)], didx1_v)
        pltpu.sync_copy(rows1_v, out_s.at[didx1_v], add=True)
        return carry

    lax.fori_loop(0, NFULL // 2, body, 0)
    off = base + NFULL * CH
    pltpu.sync_copy(src_hbm.at[pl.ds(off, REM)], idx16a_v)
    pltpu.async_copy(v_hbm.at[idx16a_v], rows16_v, sem16).wait()
    pltpu.sync_copy(dst_hbm.at[pl.ds(off, REM)], idx16b_v)
    pltpu.sync_copy(rows16_v, out_s.at[idx16b_v], add=True)
    plsc.subcore_barrier()
    pltpu.sync_copy(out_s.at[pl.ds(slab, SLAB)], out_hbm.at[c, pl.ds(slab, SLAB)])


def _edge_pass(src, dst, v, z128):
    kfn = pl.kernel(
        _pass_body,
        out_type=jax.ShapeDtypeStruct((NC, NPAD, EMB), jnp.float32),
        mesh=_sc_mesh(),
        scratch_types=[
            pltpu.VMEM_SHARED((NPAD, EMB), jnp.float32),
            pltpu.VMEM((CH,), jnp.int32),
            pltpu.VMEM((CH,), jnp.int32),
            pltpu.VMEM((CH,), jnp.int32),
            pltpu.VMEM((CH,), jnp.int32),
            pltpu.VMEM((REM,), jnp.int32),
            pltpu.VMEM((REM,), jnp.int32),
            pltpu.VMEM((CH, EMB), jnp.float32),
            pltpu.VMEM((CH, EMB), jnp.float32),
            pltpu.VMEM((REM, EMB), jnp.float32),
            pltpu.SemaphoreType.DMA,
            pltpu.SemaphoreType.DMA,
            pltpu.SemaphoreType.DMA,
        ],
    )
    return kfn(src, dst, v, z128)


def _prep0_kernel(x_ref, emb0_ref, w0t_ref, bm0_ref, degp_ref,
                  hbar_ref, v0_ref, ideg_ref, od_ref):
    rows2 = jnp.dot(emb0_ref[...], w0t_ref[...],
                    preferred_element_type=jnp.float32) + bm0_ref[...]
    hbar = jnp.where(x_ref[...] == 0, rows2[0:1, :], rows2[1:2, :])
    d = degp_ref[0] + degp_ref[1]
    od = d[:, 0:1] + 1.0
    hbar_ref[...] = hbar
    v0_ref[...] = hbar / od
    ideg_ref[...] = d[:, 64:65] + 1.0
    od_ref[...] = od


def _mid_kernel(hbar_ref, v0_ref, ideg_ref, od_ref, p_ref, bias_ref,
                w1t_ref, bm1_ref, h1_ref, v1_ref):
    comb = (ideg_ref[...] * hbar_ref[...] + v0_ref[...]
            + p_ref[0] + p_ref[1] + bias_ref[...])
    g = jnp.maximum(comb, 0.0)
    h1 = jnp.dot(g, w1t_ref[...], preferred_element_type=jnp.float32) + bm1_ref[...]
    h1_ref[...] = h1
    v1_ref[...] = h1 / od_ref[...]


def _final_kernel(h1_ref, v1_ref, ideg_ref, p_ref, bias_ref, out_ref):
    out_ref[...] = (ideg_ref[...] * h1_ref[...] + v1_ref[...]
                    + p_ref[0] + p_ref[1] + bias_ref[...])


def _row_spec():
    return pl.BlockSpec((BLK, EMB), lambda i: (i, 0))


def _col_spec():
    return pl.BlockSpec((BLK, 1), lambda i: (i, 0))


def _full_spec(shape):
    return pl.BlockSpec(shape, lambda i: tuple(0 for _ in shape))


def _part_spec():
    return pl.BlockSpec((NC, BLK, EMB), lambda i: (0, i, 0))


def _prep0(x2, emb0, w0t, bm0, degp):
    return pl.pallas_call(
        _prep0_kernel,
        grid=(N // BLK,),
        in_specs=[_col_spec(), _full_spec((2, EMB)), _full_spec((EMB, EMB)),
                  _full_spec((1, EMB)), _part_spec()],
        out_specs=[_row_spec(), _row_spec(), _col_spec(), _col_spec()],
        out_shape=[jax.ShapeDtypeStruct((N, EMB), jnp.float32),
                   jax.ShapeDtypeStruct((N, EMB), jnp.float32),
                   jax.ShapeDtypeStruct((N, 1), jnp.float32),
                   jax.ShapeDtypeStruct((N, 1), jnp.float32)],
    )(x2, emb0, w0t, bm0, degp)


def _mid(hbar, v0, ideg, od, part, bias, w1t, bm1):
    return pl.pallas_call(
        _mid_kernel,
        grid=(N // BLK,),
        in_specs=[_row_spec(), _row_spec(), _col_spec(), _col_spec(),
                  _part_spec(), _full_spec((1, EMB)),
                  _full_spec((EMB, EMB)), _full_spec((1, EMB))],
        out_specs=[_row_spec(), _row_spec()],
        out_shape=[jax.ShapeDtypeStruct((N, EMB), jnp.float32),
                   jax.ShapeDtypeStruct((N, EMB), jnp.float32)],
    )(hbar, v0, ideg, od, part, bias, w1t, bm1)


def _final(h1, v1, ideg, part, bias):
    return pl.pallas_call(
        _final_kernel,
        grid=(N // BLK,),
        in_specs=[_row_spec(), _row_spec(), _col_spec(), _part_spec(),
                  _full_spec((1, EMB))],
        out_specs=_row_spec(),
        out_shape=jax.ShapeDtypeStruct((N, EMB), jnp.float32),
    )(h1, v1, ideg, part, bias)


def kernel(x, edge_index, emb0, W0, b0, att0, bias0, W1, b1, att1, bias1):
    src = edge_index[0]
    dst = edge_index[1]
    x2 = x[:, None]

    w0t = (0.5 * (W0[:EMB] + W0[EMB:])).T
    bm0 = (0.5 * (b0[:EMB] + b0[EMB:]))[None, :]
    w1t = (0.5 * (W1[:EMB] + W1[EMB:])).T
    bm1 = (0.5 * (b1[:EMB] + b1[EMB:]))[None, :]

    z128 = jnp.zeros((NPAD, EMB), jnp.float32)
    degp = _degrees(src, dst, z128)
    hbar0, v0, ideg, od = _prep0(x2, emb0, w0t, bm0, degp)
    part0 = _edge_pass(src, dst, v0, z128)
    h1, v1 = _mid(hbar0, v0, ideg, od, part0, bias0[None, :], w1t, bm1)
    part1 = _edge_pass(src, dst, v1, z128)
    return _final(h1, v1, ideg, part1, bias1[None, :])
